# baseline (speedup 1.0000x reference)
import functools

import jax
import jax.numpy as jnp
from jax.experimental import pallas as pl
from jax.experimental.pallas import tpu as pltpu

_TG = 4096


def _fused_rowwise_kernel(x1_ref, x2_ref, w_ref, b_ref, o_ref):
    y = x1_ref[...] * w_ref[0:1, :] + x2_ref[...] * w_ref[1:2, :]
    o_ref[...] = jnp.sum(y, axis=-1, keepdims=True) + b_ref[0]


@functools.partial(jax.jit, static_argnames=("tg",))
def _fused_call(x1, x2, w, b, tg):
    B, D = x1.shape
    grid = (pl.cdiv(B, tg),)
    return pl.pallas_call(
        _fused_rowwise_kernel,
        out_shape=jax.ShapeDtypeStruct((B, 1), jnp.float32),
        grid=grid,
        in_specs=[
            pl.BlockSpec((tg, D), lambda i: (i, 0)),
            pl.BlockSpec((tg, D), lambda i: (i, 0)),
            pl.BlockSpec((2, D), lambda i: (0, 0)),
            pl.BlockSpec(memory_space=pltpu.MemorySpace.SMEM),
        ],
        out_specs=pl.BlockSpec((tg, 1), lambda i: (i, 0)),
        compiler_params=pltpu.CompilerParams(
            dimension_semantics=("parallel",),
        ),
    )(x1, x2, w, b)


def kernel(x1, x2, w1, b1, w2, b2):
    B, D = x1.shape
    w = jnp.concatenate(
        [jnp.ravel(w1).reshape(1, D), jnp.ravel(w2).reshape(1, D)], axis=0
    ).astype(jnp.float32)
    b = (jnp.ravel(b1) + jnp.ravel(b2)).astype(jnp.float32)
    tg = min(_TG, B)
    return _fused_call(x1, x2, w, b, tg)


# speedup vs baseline: 17.9124x; 17.9124x over previous
import functools

import jax
import jax.numpy as jnp
from jax.experimental import pallas as pl
from jax.experimental.pallas import tpu as pltpu

_BN = 32768


def _colwise_kernel(x1_ref, x2_ref, w1_ref, w2_ref, b_ref, o_ref):
    y = x1_ref[...] * w1_ref[...] + x2_ref[...] * w2_ref[...]
    o_ref[...] = jnp.sum(y, axis=0) + b_ref[0]


@functools.partial(jax.jit, static_argnames=("bn",))
def _colwise_call(x1t, x2t, w1c, w2c, b, bn):
    D, B = x1t.shape
    grid = (pl.cdiv(B, bn),)
    out = pl.pallas_call(
        _colwise_kernel,
        out_shape=jax.ShapeDtypeStruct((B,), jnp.float32),
        grid=grid,
        in_specs=[
            pl.BlockSpec((D, bn), lambda i: (0, i)),
            pl.BlockSpec((D, bn), lambda i: (0, i)),
            pl.BlockSpec((D, 1), lambda i: (0, 0)),
            pl.BlockSpec((D, 1), lambda i: (0, 0)),
            pl.BlockSpec(memory_space=pltpu.MemorySpace.SMEM),
        ],
        out_specs=pl.BlockSpec((bn,), lambda i: (i,)),
        compiler_params=pltpu.CompilerParams(
            dimension_semantics=("parallel",),
        ),
    )(x1t, x2t, w1c, w2c, b)
    return out.reshape(B, 1)


def kernel(x1, x2, w1, b1, w2, b2):
    B, D = x1.shape
    b = (jnp.ravel(b1) + jnp.ravel(b2)).astype(jnp.float32)
    bn = min(_BN, B)
    return _colwise_call(
        x1.T, x2.T,
        w1.reshape(D, 1).astype(jnp.float32),
        w2.reshape(D, 1).astype(jnp.float32),
        b, bn,
    )
